# baseline (speedup 1.0000x reference)
import jax
import jax.numpy as jnp
import numpy as np
from jax.experimental import pallas as pl
from jax.experimental.pallas import tpu as pltpu

INV_PI = 0.31830988618
PI_over_2 = 1.57079632679
PI_over_4 = 0.78539816339
LANES = 128
ROWS = 64
BLK = ROWS * LANES


def _body(xb, rb, w1, b1r, w2, b2r, zo, lpo):
    X = xb[...]
    H = jnp.maximum(
        jnp.dot(X, w1[...], preferred_element_type=jnp.float32) + b1r[...], 0.0)
    O = jnp.dot(H, w2[...], preferred_element_type=jnp.float32) + b2r[...]
    OT = jnp.transpose(O)
    RT = jnp.transpose(rb[...])
    o = [OT[j].reshape(ROWS, LANES) for j in range(11)]

    l00, l01, l10, l11 = o[0], o[1], o[2], o[3]
    s00, s01, s10, s11 = o[4], o[5], o[6], o[7]
    w0 = jnp.abs(o[8])
    w1_ = jnp.abs(o[9])
    w2_ = jnp.abs(o[10])
    tot = w0 + w1_ + w2_
    w0 = w0 / tot
    w1_ = w1_ / tot
    w2_ = w2_ / tot

    rdn = RT[0].reshape(ROWS, LANES)
    u2 = RT[1].reshape(ROWS, LANES)
    wc0 = w0
    wc1 = w0 + w1_
    g1 = rdn < wc0
    g2 = jnp.logical_and(~g1, rdn < wc1)
    gm = jnp.logical_or(g1, g2)
    lm = ~gm
    r0 = jnp.where(g1, rdn / wc0,
                   jnp.where(g2, (rdn - wc0) / w1_, (rdn - wc1) / w2_))

    U1 = jnp.clip(jnp.where(gm, r0, 0.5), 1e-12, 1.0 - 1e-7)
    Rbm = jnp.sqrt(-2.0 * jnp.log(U1))
    theta = 2.0 * np.pi * u2
    e0 = Rbm * jnp.cos(theta)
    e1 = Rbm * jnp.sin(theta)
    es00 = jnp.exp(s00)
    es01 = jnp.exp(s01)
    es10 = jnp.exp(s10)
    es11 = jnp.exp(s11)
    ss0 = jnp.where(g2, es10, es00)
    ss1 = jnp.where(g2, es11, es01)
    lc0 = jnp.where(g2, l10, l00)
    lc1 = jnp.where(g2, l11, l01)
    zg0 = e0 * ss0 + lc0
    zg1 = e1 * ss1 + lc1

    r0l = jnp.where(lm, r0, 0.25)
    wo0 = r0l * 2.0 - 1.0
    wo1 = u2 * 2.0 - 1.0
    zero_pos = jnp.logical_and(wo0 == 0, wo1 == 0)
    cond1 = jnp.logical_and(jnp.abs(wo0) > jnp.abs(wo1), ~zero_pos)
    cond2 = jnp.logical_and(~cond1, ~zero_pos)
    d0 = jnp.where(wo0 == 0, 1.0, wo0)
    d1 = jnp.where(wo1 == 0, 1.0, wo1)
    ang1 = PI_over_4 * wo1 / d0
    ang2 = PI_over_2 - PI_over_4 * wo0 / d1
    zl0 = jnp.where(cond1, wo0 * jnp.cos(ang1),
                    jnp.where(cond2, wo1 * jnp.cos(ang2), 0.0))
    zl1 = jnp.where(cond1, wo0 * jnp.sin(ang1),
                    jnp.where(cond2, wo1 * jnp.sin(ang2), 0.0))

    z0 = jnp.where(lm, zl0, zg0)
    z1 = jnp.where(lm, zl1, zg1)

    e_00 = (z0 - l00) / es00
    e_01 = (z1 - l01) / es01
    e_10 = (z0 - l10) / es10
    e_11 = (z1 - l11) / es11
    c = -0.5 * 2 * np.log(2.0 * np.pi)
    lg0 = c + jnp.log(w0 + 1e-5) - 0.5 * (e_00 * e_00 + e_01 * e_01) - (s00 + s01)
    lg1 = c + jnp.log(w1_ + 1e-5) - 0.5 * (e_10 * e_10 + e_11 * e_11) - (s10 + s11)
    invalid = (z0 * z0 + z1 * z1) > 1.0
    pdf = jnp.where(invalid, 0.0, INV_PI)
    ll = jnp.log(pdf + 1e-5) + jnp.log(w2_)
    m = jnp.maximum(jnp.maximum(lg0, lg1), ll)
    lp = m + jnp.log(jnp.exp(lg0 - m) + jnp.exp(lg1 - m) + jnp.exp(ll - m))

    zo[0, 0] = z0
    zo[0, 1] = z1
    lpo[0] = lp


def _build_call(G, interpret=False):
    return pl.pallas_call(
        _body,
        grid=(G,),
        in_specs=[
            pl.BlockSpec((BLK, 6), lambda i: (i, 0)),
            pl.BlockSpec((BLK, 2), lambda i: (i, 0)),
            pl.BlockSpec((6, 32), lambda i: (0, 0)),
            pl.BlockSpec((1, 32), lambda i: (0, 0)),
            pl.BlockSpec((32, 11), lambda i: (0, 0)),
            pl.BlockSpec((1, 11), lambda i: (0, 0)),
        ],
        out_specs=[
            pl.BlockSpec((1, 2, ROWS, LANES), lambda i: (i, 0, 0, 0)),
            pl.BlockSpec((1, ROWS, LANES), lambda i: (i, 0, 0)),
        ],
        out_shape=[
            jax.ShapeDtypeStruct((G, 2, ROWS, LANES), jnp.float32),
            jax.ShapeDtypeStruct((G, ROWS, LANES), jnp.float32),
        ],
        interpret=interpret,
    )


def kernel(cond_vec, randseed, W1, b1, W2, b2, num_samples):
    n = cond_vec.shape[0]
    G = n // BLK
    zT, lp = _build_call(G)(cond_vec, randseed, W1, b1.reshape(1, 32),
                            W2, b2.reshape(1, 11))
    z = zT.transpose(1, 0, 2, 3).reshape(2, n).T
    logp = lp.reshape(n)
    return z, logp


# speedup vs baseline: 4.9985x; 4.9985x over previous
import jax
import jax.numpy as jnp
import numpy as np
from jax.experimental import pallas as pl
from jax.experimental.pallas import tpu as pltpu

INV_PI = 0.31830988618
PI_over_2 = 1.57079632679
PI_over_4 = 0.78539816339
LANES = 128
ROWS = 64
BLK = ROWS * LANES


def _body(xT, rT, w1t, b1c, w2t, b2c, zo, lpo):
    X = xT[0]
    H = jnp.maximum(
        jnp.dot(w1t[...], X, preferred_element_type=jnp.float32) + b1c[...], 0.0)
    O = jnp.dot(w2t[...], H, preferred_element_type=jnp.float32) + b2c[...]
    o = [O[j].reshape(ROWS, LANES) for j in range(11)]

    l00, l01, l10, l11 = o[0], o[1], o[2], o[3]
    s00, s01, s10, s11 = o[4], o[5], o[6], o[7]
    w0 = jnp.abs(o[8])
    w1 = jnp.abs(o[9])
    w2 = jnp.abs(o[10])
    tot = w0 + w1 + w2
    w0 = w0 / tot
    w1 = w1 / tot
    w2 = w2 / tot

    rdn = rT[0, 0]
    u2 = rT[0, 1]
    wc0 = w0
    wc1 = w0 + w1
    g1 = rdn < wc0
    g2 = jnp.logical_and(~g1, rdn < wc1)
    gm = jnp.logical_or(g1, g2)
    lm = ~gm
    r0 = jnp.where(g1, rdn / wc0,
                   jnp.where(g2, (rdn - wc0) / w1, (rdn - wc1) / w2))

    U1 = jnp.clip(jnp.where(gm, r0, 0.5), 1e-12, 1.0 - 1e-7)
    Rbm = jnp.sqrt(-2.0 * jnp.log(U1))
    theta = 2.0 * np.pi * u2
    e0 = Rbm * jnp.cos(theta)
    e1 = Rbm * jnp.sin(theta)
    es00 = jnp.exp(s00)
    es01 = jnp.exp(s01)
    es10 = jnp.exp(s10)
    es11 = jnp.exp(s11)
    ss0 = jnp.where(g2, es10, es00)
    ss1 = jnp.where(g2, es11, es01)
    lc0 = jnp.where(g2, l10, l00)
    lc1 = jnp.where(g2, l11, l01)
    zg0 = e0 * ss0 + lc0
    zg1 = e1 * ss1 + lc1

    r0l = jnp.where(lm, r0, 0.25)
    wo0 = r0l * 2.0 - 1.0
    wo1 = u2 * 2.0 - 1.0
    zero_pos = jnp.logical_and(wo0 == 0, wo1 == 0)
    cond1 = jnp.logical_and(jnp.abs(wo0) > jnp.abs(wo1), ~zero_pos)
    d0 = jnp.where(wo0 == 0, 1.0, wo0)
    d1 = jnp.where(wo1 == 0, 1.0, wo1)
    ang1 = PI_over_4 * wo1 / d0
    ang2 = PI_over_2 - PI_over_4 * wo0 / d1
    rad = jnp.where(cond1, wo0, wo1)
    ang = jnp.where(cond1, ang1, ang2)
    zl0 = jnp.where(zero_pos, 0.0, rad * jnp.cos(ang))
    zl1 = jnp.where(zero_pos, 0.0, rad * jnp.sin(ang))

    z0 = jnp.where(lm, zl0, zg0)
    z1 = jnp.where(lm, zl1, zg1)

    e_00 = (z0 - l00) / es00
    e_01 = (z1 - l01) / es01
    e_10 = (z0 - l10) / es10
    e_11 = (z1 - l11) / es11
    c = -0.5 * 2 * np.log(2.0 * np.pi)
    lg0 = c + jnp.log(w0 + 1e-5) - 0.5 * (e_00 * e_00 + e_01 * e_01) - (s00 + s01)
    lg1 = c + jnp.log(w1 + 1e-5) - 0.5 * (e_10 * e_10 + e_11 * e_11) - (s10 + s11)
    invalid = (z0 * z0 + z1 * z1) > 1.0
    ll = jnp.where(invalid, np.float32(np.log(np.float32(1e-5))),
                   np.float32(np.log(np.float32(INV_PI) + np.float32(1e-5))))
    ll = ll + jnp.log(w2)
    m = jnp.maximum(jnp.maximum(lg0, lg1), ll)
    lp = m + jnp.log(jnp.exp(lg0 - m) + jnp.exp(lg1 - m) + jnp.exp(ll - m))

    zo[0, 0] = z0
    zo[0, 1] = z1
    lpo[0] = lp


def _build_call(G, interpret=False):
    return pl.pallas_call(
        _body,
        grid=(G,),
        in_specs=[
            pl.BlockSpec((1, 6, BLK), lambda i: (i, 0, 0)),
            pl.BlockSpec((1, 2, ROWS, LANES), lambda i: (i, 0, 0, 0)),
            pl.BlockSpec((32, 6), lambda i: (0, 0)),
            pl.BlockSpec((32, 1), lambda i: (0, 0)),
            pl.BlockSpec((11, 32), lambda i: (0, 0)),
            pl.BlockSpec((11, 1), lambda i: (0, 0)),
        ],
        out_specs=[
            pl.BlockSpec((1, 2, ROWS, LANES), lambda i: (i, 0, 0, 0)),
            pl.BlockSpec((1, ROWS, LANES), lambda i: (i, 0, 0)),
        ],
        out_shape=[
            jax.ShapeDtypeStruct((G, 2, ROWS, LANES), jnp.float32),
            jax.ShapeDtypeStruct((G, ROWS, LANES), jnp.float32),
        ],
        interpret=interpret,
    )


def kernel(cond_vec, randseed, W1, b1, W2, b2, num_samples):
    n = cond_vec.shape[0]
    G = n // BLK
    xT = cond_vec.T.reshape(6, G, BLK).transpose(1, 0, 2)
    rT = randseed.T.reshape(2, G, ROWS, LANES).transpose(1, 0, 2, 3)
    zT, lp = _build_call(G)(xT, rT, W1.T, b1.reshape(32, 1),
                            W2.T, b2.reshape(11, 1))
    z = zT.transpose(1, 0, 2, 3).reshape(2, n).T
    logp = lp.reshape(n)
    return z, logp


# speedup vs baseline: 5.1792x; 1.0362x over previous
import jax
import jax.numpy as jnp
import numpy as np
from jax.experimental import pallas as pl
from jax.experimental.pallas import tpu as pltpu

INV_PI = 0.31830988618
PI_over_2 = 1.57079632679
PI_over_4 = 0.78539816339
LANES = 128
ROWS = 64
BLK = ROWS * LANES


def _body(xT, rT, w1t, b1c, w2t, b2c, zo, lpo):
    X = xT[0]
    H = jnp.maximum(
        jnp.dot(w1t[...], X, preferred_element_type=jnp.float32) + b1c[...], 0.0)
    O = jnp.dot(w2t[...], H, preferred_element_type=jnp.float32) + b2c[...]
    o = [O[j].reshape(ROWS, LANES) for j in range(11)]

    l00, l01, l10, l11 = o[0], o[1], o[2], o[3]
    s00, s01, s10, s11 = o[4], o[5], o[6], o[7]
    w0 = jnp.abs(o[8])
    w1 = jnp.abs(o[9])
    w2 = jnp.abs(o[10])
    tot = w0 + w1 + w2
    w0 = w0 / tot
    w1 = w1 / tot
    w2 = w2 / tot

    rdn = rT[0, 0]
    u2 = rT[0, 1]
    wc0 = w0
    wc1 = w0 + w1
    g1 = rdn < wc0
    g2 = jnp.logical_and(~g1, rdn < wc1)
    gm = jnp.logical_or(g1, g2)
    lm = ~gm
    num = jnp.where(g1, rdn, jnp.where(g2, rdn - wc0, rdn - wc1))
    den = jnp.where(g1, wc0, jnp.where(g2, w1, w2))
    r0 = num / den

    U1 = jnp.clip(jnp.where(gm, r0, 0.5), 1e-12, 1.0 - 1e-7)
    Rbm = jnp.sqrt(-2.0 * jnp.log(U1))
    theta = 2.0 * np.pi * u2
    e0 = Rbm * jnp.cos(theta)
    e1 = Rbm * jnp.sin(theta)
    es00 = jnp.exp(s00)
    es01 = jnp.exp(s01)
    es10 = jnp.exp(s10)
    es11 = jnp.exp(s11)
    ss0 = jnp.where(g2, es10, es00)
    ss1 = jnp.where(g2, es11, es01)
    lc0 = jnp.where(g2, l10, l00)
    lc1 = jnp.where(g2, l11, l01)
    zg0 = e0 * ss0 + lc0
    zg1 = e1 * ss1 + lc1

    r0l = jnp.where(lm, r0, 0.25)
    wo0 = r0l * 2.0 - 1.0
    wo1 = u2 * 2.0 - 1.0
    zero_pos = jnp.logical_and(wo0 == 0, wo1 == 0)
    cond1 = jnp.logical_and(jnp.abs(wo0) > jnp.abs(wo1), ~zero_pos)
    d0 = jnp.where(wo0 == 0, 1.0, wo0)
    d1 = jnp.where(wo1 == 0, 1.0, wo1)
    anum = jnp.where(cond1, wo1, wo0)
    aden = jnp.where(cond1, d0, d1)
    t = (PI_over_4 * anum) / aden
    rad = jnp.where(cond1, wo0, wo1)
    ang = jnp.where(cond1, t, PI_over_2 - t)
    zl0 = jnp.where(zero_pos, 0.0, rad * jnp.cos(ang))
    zl1 = jnp.where(zero_pos, 0.0, rad * jnp.sin(ang))

    z0 = jnp.where(lm, zl0, zg0)
    z1 = jnp.where(lm, zl1, zg1)

    e_00 = (z0 - l00) * jnp.exp(-s00)
    e_01 = (z1 - l01) * jnp.exp(-s01)
    e_10 = (z0 - l10) * jnp.exp(-s10)
    e_11 = (z1 - l11) * jnp.exp(-s11)
    c = -0.5 * 2 * np.log(2.0 * np.pi)
    lg0 = c + jnp.log(w0 + 1e-5) - 0.5 * (e_00 * e_00 + e_01 * e_01) - (s00 + s01)
    lg1 = c + jnp.log(w1 + 1e-5) - 0.5 * (e_10 * e_10 + e_11 * e_11) - (s10 + s11)
    invalid = (z0 * z0 + z1 * z1) > 1.0
    ll = jnp.where(invalid, np.float32(np.log(np.float32(1e-5))),
                   np.float32(np.log(np.float32(INV_PI) + np.float32(1e-5))))
    ll = ll + jnp.log(w2)
    m = jnp.maximum(jnp.maximum(lg0, lg1), ll)
    lp = m + jnp.log(jnp.exp(lg0 - m) + jnp.exp(lg1 - m) + jnp.exp(ll - m))

    zo[0, 0] = z0
    zo[0, 1] = z1
    lpo[0] = lp


def _build_call(G, interpret=False):
    return pl.pallas_call(
        _body,
        grid=(G,),
        in_specs=[
            pl.BlockSpec((1, 6, BLK), lambda i: (i, 0, 0)),
            pl.BlockSpec((1, 2, ROWS, LANES), lambda i: (i, 0, 0, 0)),
            pl.BlockSpec((32, 6), lambda i: (0, 0)),
            pl.BlockSpec((32, 1), lambda i: (0, 0)),
            pl.BlockSpec((11, 32), lambda i: (0, 0)),
            pl.BlockSpec((11, 1), lambda i: (0, 0)),
        ],
        out_specs=[
            pl.BlockSpec((1, 2, ROWS, LANES), lambda i: (i, 0, 0, 0)),
            pl.BlockSpec((1, ROWS, LANES), lambda i: (i, 0, 0)),
        ],
        out_shape=[
            jax.ShapeDtypeStruct((G, 2, ROWS, LANES), jnp.float32),
            jax.ShapeDtypeStruct((G, ROWS, LANES), jnp.float32),
        ],
        interpret=interpret,
    )


def kernel(cond_vec, randseed, W1, b1, W2, b2, num_samples):
    n = cond_vec.shape[0]
    G = n // BLK
    xT = cond_vec.T.reshape(6, G, BLK).transpose(1, 0, 2)
    rT = randseed.T.reshape(2, G, ROWS, LANES).transpose(1, 0, 2, 3)
    zT, lp = _build_call(G)(xT, rT, W1.T, b1.reshape(32, 1),
                            W2.T, b2.reshape(11, 1))
    z = zT.transpose(1, 0, 2, 3).reshape(2, n).T
    logp = lp.reshape(n)
    return z, logp


# speedup vs baseline: 6.3935x; 1.2345x over previous
import jax
import jax.numpy as jnp
import numpy as np
from jax.experimental import pallas as pl
from jax.experimental.pallas import tpu as pltpu

INV_PI = 0.31830988618
PI_over_2 = 1.57079632679
PI_over_4 = 0.78539816339
LANES = 128
ROWS = 128
BLK = ROWS * LANES


def _body(xT, rT, w1t, b1c, w2t, b2c, zo, lpo):
    X = xT[0]
    H = jnp.maximum(
        jnp.dot(w1t[...], X, preferred_element_type=jnp.float32) + b1c[...], 0.0)
    O = jnp.dot(w2t[...], H, preferred_element_type=jnp.float32) + b2c[...]
    o = [O[j].reshape(ROWS, LANES) for j in range(11)]

    l00, l01, l10, l11 = o[0], o[1], o[2], o[3]
    s00, s01, s10, s11 = o[4], o[5], o[6], o[7]
    w0 = jnp.abs(o[8])
    w1 = jnp.abs(o[9])
    w2 = jnp.abs(o[10])
    tot = w0 + w1 + w2
    w0 = w0 / tot
    w1 = w1 / tot
    w2 = w2 / tot

    rdn = rT[0, 0]
    u2 = rT[0, 1]
    wc0 = w0
    wc1 = w0 + w1
    g1 = rdn < wc0
    g2 = jnp.logical_and(~g1, rdn < wc1)
    gm = jnp.logical_or(g1, g2)
    lm = ~gm
    num = jnp.where(g1, rdn, jnp.where(g2, rdn - wc0, rdn - wc1))
    den = jnp.where(g1, wc0, jnp.where(g2, w1, w2))
    r0 = num / den

    U1 = jnp.clip(jnp.where(gm, r0, 0.5), 1e-12, 1.0 - 1e-7)
    Rbm = jnp.sqrt(-2.0 * jnp.log(U1))
    theta = 2.0 * np.pi * u2
    es00 = jnp.exp(s00)
    es01 = jnp.exp(s01)
    es10 = jnp.exp(s10)
    es11 = jnp.exp(s11)
    ss0 = jnp.where(g2, es10, es00)
    ss1 = jnp.where(g2, es11, es01)
    lc0 = jnp.where(g2, l10, l00)
    lc1 = jnp.where(g2, l11, l01)

    r0l = jnp.where(lm, r0, 0.25)
    wo0 = r0l * 2.0 - 1.0
    wo1 = u2 * 2.0 - 1.0
    zero_pos = jnp.logical_and(wo0 == 0, wo1 == 0)
    cond1 = jnp.logical_and(jnp.abs(wo0) > jnp.abs(wo1), ~zero_pos)
    d0 = jnp.where(wo0 == 0, 1.0, wo0)
    d1 = jnp.where(wo1 == 0, 1.0, wo1)
    anum = jnp.where(cond1, wo1, wo0)
    aden = jnp.where(cond1, d0, d1)
    t = (PI_over_4 * anum) / aden
    rad = jnp.where(cond1, wo0, wo1)
    ang = jnp.where(cond1, t, PI_over_2 - t)

    angle = jnp.where(lm, ang, theta)
    ca = jnp.cos(angle)
    sa = jnp.sin(angle)
    zg0 = Rbm * ca * ss0 + lc0
    zg1 = Rbm * sa * ss1 + lc1
    zl0 = jnp.where(zero_pos, 0.0, rad * ca)
    zl1 = jnp.where(zero_pos, 0.0, rad * sa)

    z0 = jnp.where(lm, zl0, zg0)
    z1 = jnp.where(lm, zl1, zg1)

    e_00 = (z0 - l00) * jnp.exp(-s00)
    e_01 = (z1 - l01) * jnp.exp(-s01)
    e_10 = (z0 - l10) * jnp.exp(-s10)
    e_11 = (z1 - l11) * jnp.exp(-s11)
    c = -0.5 * 2 * np.log(2.0 * np.pi)
    lg0 = c + jnp.log(w0 + 1e-5) - 0.5 * (e_00 * e_00 + e_01 * e_01) - (s00 + s01)
    lg1 = c + jnp.log(w1 + 1e-5) - 0.5 * (e_10 * e_10 + e_11 * e_11) - (s10 + s11)
    invalid = (z0 * z0 + z1 * z1) > 1.0
    ll = jnp.where(invalid, np.float32(np.log(np.float32(1e-5))),
                   np.float32(np.log(np.float32(INV_PI) + np.float32(1e-5))))
    ll = ll + jnp.log(w2)
    m = jnp.maximum(jnp.maximum(lg0, lg1), ll)
    lp = m + jnp.log(jnp.exp(lg0 - m) + jnp.exp(lg1 - m) + jnp.exp(ll - m))

    zo[0, 0] = z0
    zo[0, 1] = z1
    lpo[0] = lp


def _build_call(G, interpret=False):
    return pl.pallas_call(
        _body,
        grid=(G,),
        in_specs=[
            pl.BlockSpec((1, 6, BLK), lambda i: (i, 0, 0)),
            pl.BlockSpec((1, 2, ROWS, LANES), lambda i: (i, 0, 0, 0)),
            pl.BlockSpec((32, 6), lambda i: (0, 0)),
            pl.BlockSpec((32, 1), lambda i: (0, 0)),
            pl.BlockSpec((11, 32), lambda i: (0, 0)),
            pl.BlockSpec((11, 1), lambda i: (0, 0)),
        ],
        out_specs=[
            pl.BlockSpec((1, 2, ROWS, LANES), lambda i: (i, 0, 0, 0)),
            pl.BlockSpec((1, ROWS, LANES), lambda i: (i, 0, 0)),
        ],
        out_shape=[
            jax.ShapeDtypeStruct((G, 2, ROWS, LANES), jnp.float32),
            jax.ShapeDtypeStruct((G, ROWS, LANES), jnp.float32),
        ],
        interpret=interpret,
    )


def kernel(cond_vec, randseed, W1, b1, W2, b2, num_samples):
    n = cond_vec.shape[0]
    G = n // BLK
    xT = cond_vec.T.reshape(6, G, BLK).transpose(1, 0, 2)
    rT = randseed.T.reshape(2, G, ROWS, LANES).transpose(1, 0, 2, 3)
    zT, lp = _build_call(G)(xT, rT, W1.T, b1.reshape(32, 1),
                            W2.T, b2.reshape(11, 1))
    z = zT.transpose(1, 0, 2, 3).reshape(2, n).T
    logp = lp.reshape(n)
    return z, logp


# speedup vs baseline: 6.5558x; 1.0254x over previous
import jax
import jax.numpy as jnp
import numpy as np
from jax.experimental import pallas as pl
from jax.experimental.pallas import tpu as pltpu

INV_PI = 0.31830988618
PI_over_2 = 1.57079632679
PI_over_4 = 0.78539816339
LANES = 128
ROWS = 256
BLK = ROWS * LANES


def _body(xT, rT, w1t, b1c, w2t, b2c, zo, lpo):
    X = xT[0]
    H = jnp.maximum(
        jnp.dot(w1t[...], X, preferred_element_type=jnp.float32) + b1c[...], 0.0)
    O = jnp.dot(w2t[...], H, preferred_element_type=jnp.float32) + b2c[...]
    o = [O[j].reshape(ROWS, LANES) for j in range(11)]

    l00, l01, l10, l11 = o[0], o[1], o[2], o[3]
    s00, s01, s10, s11 = o[4], o[5], o[6], o[7]
    w0 = jnp.abs(o[8])
    w1 = jnp.abs(o[9])
    w2 = jnp.abs(o[10])
    tot = w0 + w1 + w2
    w0 = w0 / tot
    w1 = w1 / tot
    w2 = w2 / tot

    rdn = rT[0, 0]
    u2 = rT[0, 1]
    wc0 = w0
    wc1 = w0 + w1
    g1 = rdn < wc0
    g2 = jnp.logical_and(~g1, rdn < wc1)
    gm = jnp.logical_or(g1, g2)
    lm = ~gm
    num = jnp.where(g1, rdn, jnp.where(g2, rdn - wc0, rdn - wc1))
    den = jnp.where(g1, wc0, jnp.where(g2, w1, w2))
    r0 = num / den

    U1 = jnp.clip(jnp.where(gm, r0, 0.5), 1e-12, 1.0 - 1e-7)
    Rbm = jnp.sqrt(-2.0 * jnp.log(U1))
    theta = 2.0 * np.pi * u2
    es00 = jnp.exp(s00)
    es01 = jnp.exp(s01)
    es10 = jnp.exp(s10)
    es11 = jnp.exp(s11)
    ss0 = jnp.where(g2, es10, es00)
    ss1 = jnp.where(g2, es11, es01)
    lc0 = jnp.where(g2, l10, l00)
    lc1 = jnp.where(g2, l11, l01)

    r0l = jnp.where(lm, r0, 0.25)
    wo0 = r0l * 2.0 - 1.0
    wo1 = u2 * 2.0 - 1.0
    zero_pos = jnp.logical_and(wo0 == 0, wo1 == 0)
    cond1 = jnp.logical_and(jnp.abs(wo0) > jnp.abs(wo1), ~zero_pos)
    d0 = jnp.where(wo0 == 0, 1.0, wo0)
    d1 = jnp.where(wo1 == 0, 1.0, wo1)
    anum = jnp.where(cond1, wo1, wo0)
    aden = jnp.where(cond1, d0, d1)
    t = (PI_over_4 * anum) / aden
    rad = jnp.where(cond1, wo0, wo1)
    ang = jnp.where(cond1, t, PI_over_2 - t)

    angle = jnp.where(lm, ang, theta)
    ca = jnp.cos(angle)
    sa = jnp.sin(angle)
    zg0 = Rbm * ca * ss0 + lc0
    zg1 = Rbm * sa * ss1 + lc1
    zl0 = jnp.where(zero_pos, 0.0, rad * ca)
    zl1 = jnp.where(zero_pos, 0.0, rad * sa)

    z0 = jnp.where(lm, zl0, zg0)
    z1 = jnp.where(lm, zl1, zg1)

    e_00 = (z0 - l00) * jnp.exp(-s00)
    e_01 = (z1 - l01) * jnp.exp(-s01)
    e_10 = (z0 - l10) * jnp.exp(-s10)
    e_11 = (z1 - l11) * jnp.exp(-s11)
    c = -0.5 * 2 * np.log(2.0 * np.pi)
    lg0 = c + jnp.log(w0 + 1e-5) - 0.5 * (e_00 * e_00 + e_01 * e_01) - (s00 + s01)
    lg1 = c + jnp.log(w1 + 1e-5) - 0.5 * (e_10 * e_10 + e_11 * e_11) - (s10 + s11)
    invalid = (z0 * z0 + z1 * z1) > 1.0
    ll = jnp.where(invalid, np.float32(np.log(np.float32(1e-5))),
                   np.float32(np.log(np.float32(INV_PI) + np.float32(1e-5))))
    ll = ll + jnp.log(w2)
    m = jnp.maximum(jnp.maximum(lg0, lg1), ll)
    lp = m + jnp.log(jnp.exp(lg0 - m) + jnp.exp(lg1 - m) + jnp.exp(ll - m))

    zo[0, 0] = z0
    zo[0, 1] = z1
    lpo[0] = lp


def _build_call(G, interpret=False):
    return pl.pallas_call(
        _body,
        grid=(G,),
        in_specs=[
            pl.BlockSpec((1, 6, BLK), lambda i: (i, 0, 0)),
            pl.BlockSpec((1, 2, ROWS, LANES), lambda i: (i, 0, 0, 0)),
            pl.BlockSpec((32, 6), lambda i: (0, 0)),
            pl.BlockSpec((32, 1), lambda i: (0, 0)),
            pl.BlockSpec((11, 32), lambda i: (0, 0)),
            pl.BlockSpec((11, 1), lambda i: (0, 0)),
        ],
        out_specs=[
            pl.BlockSpec((1, 2, ROWS, LANES), lambda i: (i, 0, 0, 0)),
            pl.BlockSpec((1, ROWS, LANES), lambda i: (i, 0, 0)),
        ],
        out_shape=[
            jax.ShapeDtypeStruct((G, 2, ROWS, LANES), jnp.float32),
            jax.ShapeDtypeStruct((G, ROWS, LANES), jnp.float32),
        ],
        interpret=interpret,
    )


def kernel(cond_vec, randseed, W1, b1, W2, b2, num_samples):
    n = cond_vec.shape[0]
    G = n // BLK
    xT = cond_vec.T.reshape(6, G, BLK).transpose(1, 0, 2)
    rT = randseed.T.reshape(2, G, ROWS, LANES).transpose(1, 0, 2, 3)
    zT, lp = _build_call(G)(xT, rT, W1.T, b1.reshape(32, 1),
                            W2.T, b2.reshape(11, 1))
    z = zT.transpose(1, 0, 2, 3).reshape(2, n).T
    logp = lp.reshape(n)
    return z, logp


# speedup vs baseline: 6.5572x; 1.0002x over previous
import jax
import jax.numpy as jnp
import numpy as np
from jax.experimental import pallas as pl
from jax.experimental.pallas import tpu as pltpu

INV_PI = 0.31830988618
PI_over_2 = 1.57079632679
PI_over_4 = 0.78539816339
LANES = 128
ROWS = 256
BLK = ROWS * LANES


def _body(xT, rT, w1t, b1c, w2t, b2c, zo, lpo):
    X = xT[0]
    H = jnp.maximum(
        jnp.dot(w1t[...], X, preferred_element_type=jnp.float32) + b1c[...], 0.0)
    O = jnp.dot(w2t[...], H, preferred_element_type=jnp.float32) + b2c[...]
    o = [O[j].reshape(ROWS, LANES) for j in range(11)]

    l00, l01, l10, l11 = o[0], o[1], o[2], o[3]
    s00, s01, s10, s11 = o[4], o[5], o[6], o[7]
    w0 = jnp.abs(o[8])
    w1 = jnp.abs(o[9])
    w2 = jnp.abs(o[10])
    tot = w0 + w1 + w2
    w0 = w0 / tot
    w1 = w1 / tot
    w2 = w2 / tot

    rdn = rT[0, 0]
    u2 = rT[0, 1]
    wc0 = w0
    wc1 = w0 + w1
    g1 = rdn < wc0
    g2 = jnp.logical_and(~g1, rdn < wc1)
    gm = jnp.logical_or(g1, g2)
    lm = ~gm
    num = jnp.where(g1, rdn, jnp.where(g2, rdn - wc0, rdn - wc1))
    den = jnp.where(g1, wc0, jnp.where(g2, w1, w2))
    r0 = num / den

    U1 = jnp.clip(jnp.where(gm, r0, 0.5), 1e-12, 1.0 - 1e-7)
    Rbm = jnp.sqrt(-2.0 * jnp.log(U1))
    theta = 2.0 * np.pi * u2
    es00 = jnp.exp(s00)
    es01 = jnp.exp(s01)
    es10 = jnp.exp(s10)
    es11 = jnp.exp(s11)
    ss0 = jnp.where(g2, es10, es00)
    ss1 = jnp.where(g2, es11, es01)
    lc0 = jnp.where(g2, l10, l00)
    lc1 = jnp.where(g2, l11, l01)

    r0l = jnp.where(lm, r0, 0.25)
    wo0 = r0l * 2.0 - 1.0
    wo1 = u2 * 2.0 - 1.0
    zero_pos = jnp.logical_and(wo0 == 0, wo1 == 0)
    cond1 = jnp.logical_and(jnp.abs(wo0) > jnp.abs(wo1), ~zero_pos)
    d0 = jnp.where(wo0 == 0, 1.0, wo0)
    d1 = jnp.where(wo1 == 0, 1.0, wo1)
    anum = jnp.where(cond1, wo1, wo0)
    aden = jnp.where(cond1, d0, d1)
    t = (PI_over_4 * anum) / aden
    rad = jnp.where(cond1, wo0, wo1)
    ang = jnp.where(cond1, t, PI_over_2 - t)

    angle = jnp.where(lm, ang, theta)
    ca = jnp.cos(angle)
    sa = jnp.sin(angle)
    zg0 = Rbm * ca * ss0 + lc0
    zg1 = Rbm * sa * ss1 + lc1
    zl0 = jnp.where(zero_pos, 0.0, rad * ca)
    zl1 = jnp.where(zero_pos, 0.0, rad * sa)

    z0 = jnp.where(lm, zl0, zg0)
    z1 = jnp.where(lm, zl1, zg1)

    e_00 = (z0 - l00) * jnp.exp(-s00)
    e_01 = (z1 - l01) * jnp.exp(-s01)
    e_10 = (z0 - l10) * jnp.exp(-s10)
    e_11 = (z1 - l11) * jnp.exp(-s11)
    c = -0.5 * 2 * np.log(2.0 * np.pi)
    lg0 = c + jnp.log(w0 + 1e-5) - 0.5 * (e_00 * e_00 + e_01 * e_01) - (s00 + s01)
    lg1 = c + jnp.log(w1 + 1e-5) - 0.5 * (e_10 * e_10 + e_11 * e_11) - (s10 + s11)
    invalid = (z0 * z0 + z1 * z1) > 1.0
    ll = jnp.where(invalid, np.float32(np.log(np.float32(1e-5))),
                   np.float32(np.log(np.float32(INV_PI) + np.float32(1e-5))))
    ll = ll + jnp.log(w2)
    m = jnp.maximum(jnp.maximum(lg0, lg1), ll)
    lp = m + jnp.log(jnp.exp(lg0 - m) + jnp.exp(lg1 - m) + jnp.exp(ll - m))

    zo[0, 0] = z0
    zo[0, 1] = z1
    lpo[0] = lp


def _build_call(G):
    return pl.pallas_call(
        _body,
        grid=(G,),
        in_specs=[
            pl.BlockSpec((1, 6, BLK), lambda i: (i, 0, 0)),
            pl.BlockSpec((1, 2, ROWS, LANES), lambda i: (i, 0, 0, 0)),
            pl.BlockSpec((32, 6), lambda i: (0, 0)),
            pl.BlockSpec((32, 1), lambda i: (0, 0)),
            pl.BlockSpec((11, 32), lambda i: (0, 0)),
            pl.BlockSpec((11, 1), lambda i: (0, 0)),
        ],
        out_specs=[
            pl.BlockSpec((1, 2, ROWS, LANES), lambda i: (i, 0, 0, 0)),
            pl.BlockSpec((1, ROWS, LANES), lambda i: (i, 0, 0)),
        ],
        out_shape=[
            jax.ShapeDtypeStruct((G, 2, ROWS, LANES), jnp.float32),
            jax.ShapeDtypeStruct((G, ROWS, LANES), jnp.float32),
        ],
    )


def kernel(cond_vec, randseed, W1, b1, W2, b2, num_samples):
    n = cond_vec.shape[0]
    G = n // BLK
    xT = cond_vec.T.reshape(6, G, BLK).transpose(1, 0, 2)
    rT = randseed.T.reshape(2, G, ROWS, LANES).transpose(1, 0, 2, 3)
    zT, lp = _build_call(G)(xT, rT, W1.T, b1.reshape(32, 1),
                            W2.T, b2.reshape(11, 1))
    z = zT.transpose(1, 0, 2, 3).reshape(2, n).T
    logp = lp.reshape(n)
    return z, logp
